# Initial kernel scaffold; baseline (speedup 1.0000x reference)
import jax
import jax.numpy as jnp
from jax.experimental import pallas as pl


def kernel(unseen_entity, triplets, entity_embedding, relation_embedding, basis_mu, att_mu, root_mu, bias_mu, basis_sigma, att_sigma, root_sigma, bias_sigma, total_unseen_entity_embedding):
    raise NotImplementedError("write your pallas kernel here")


# speedup vs baseline: 11.3484x; 11.3484x over previous
import functools

import jax
import jax.numpy as jnp
from jax import lax
from jax.experimental import pallas as pl
from jax.experimental.pallas import tpu as pltpu
from jax.experimental.pallas import tpu_sc as plsc

N_ENT = 50000
N_REL = 200
D = 128
T = 100000
U = 500

NC = 2
NS = 16
NW = NC * NS
C = 3136
TP = NW * C
G = C // 16
CAP = 6656
PM = 50016
UP = 512
ROWS = 512
RPT = ROWS // NS
W = 4 * D + 2 * D + 16
CH = 512
PAD_ENT = N_ENT


def _sc_kernel_body(srcp, relp, dstp, unseen, attmu, attsg, xcat, rtab,
                    out,
                    posmap, src_v, rel_v, dst_v, unseen_v, attmu_v, attsg_v,
                    gidx_l, et_l, row_l, cntbuf, cnt_all,
                    gch, ech, rch, loc_g, loc_e, loc_r,
                    gbuf, ebuf, xrows, rrows, acc,
                    stage_g, stage_e, stage_r, cstage, sem1, sem2):
    cid = lax.axis_index("c")
    sid = lax.axis_index("s")
    wid = sid * NC + cid
    lane = lax.iota(jnp.int32, 16)
    zeros16 = jnp.zeros((16,), jnp.int32)
    zerosf = jnp.zeros((16,), jnp.float32)
    wrow = jnp.where(lane == 0, 1.0, 0.0)

    pltpu.sync_copy(unseen.at[pl.ds(0, UP)], unseen_v)
    pltpu.sync_copy(attmu, attmu_v)
    pltpu.sync_copy(attsg, attsg_v)
    base_t = wid * C
    pltpu.sync_copy(srcp.at[pl.ds(base_t, C)], src_v)
    pltpu.sync_copy(relp.at[pl.ds(base_t, C)], rel_v)
    pltpu.sync_copy(dstp.at[pl.ds(base_t, C)], dst_v)

    def _zero_pm(i, _):
        posmap[pl.ds(i * 16, 16)] = zeros16
        return 0
    lax.fori_loop(0, PM // 16, _zero_pm, 0)

    def _build_pm(j, _):
        u = unseen_v[pl.ds(j * 16, 16)]
        pos = j * 16 + lane
        plsc.store_scatter(posmap, [u], pos + 1, mask=pos < U)
        return 0
    lax.fori_loop(0, UP // 16, _build_pm, 0)

    def _zero_acc(r, _):
        for k in range(W // 16):
            acc[r, pl.ds(k * 16, 16)] = zerosf
        return 0
    lax.fori_loop(0, RPT, _zero_acc, 0)

    def _zero_loc(i, _):
        sl = pl.ds(i * 16, 16)
        loc_g[sl] = zeros16
        loc_e[sl] = zeros16
        loc_r[sl] = zeros16
        return 0
    lax.fori_loop(0, CH // 16, _zero_loc, 0)

    def _scan(g, off):
        sl = pl.ds(g * 16, 16)
        s = src_v[sl]
        r = rel_v[sl]
        d = dst_v[sl]
        pd = plsc.load_gather(posmap, [d])
        ps = plsc.load_gather(posmap, [s])
        md = pd > 0
        ms = ps > 0
        cf = plsc.cumsum(jnp.where(md, 1, 0))
        pf = off + cf - 1
        gi_f = jnp.where(ms, ps + (N_ENT - 1), s)
        plsc.store_scatter(gidx_l, [pf], gi_f, mask=md)
        plsc.store_scatter(et_l, [pf], r, mask=md)
        plsc.store_scatter(row_l, [pf], pd - 1, mask=md)
        off = off + jnp.sum(jnp.where(md, 1, 0))
        cr = plsc.cumsum(jnp.where(ms, 1, 0))
        pr = off + cr - 1
        gi_r = jnp.where(md, pd + (N_ENT - 1), d)
        plsc.store_scatter(gidx_l, [pr], gi_r, mask=ms)
        plsc.store_scatter(et_l, [pr], r + N_REL, mask=ms)
        plsc.store_scatter(row_l, [pr], ps - 1, mask=ms)
        off = off + jnp.sum(jnp.where(ms, 1, 0))
        return off
    n_edges = lax.fori_loop(0, G, _scan, jnp.int32(0))

    cntbuf[...] = zeros16 + n_edges
    pltpu.sync_copy(cntbuf, cstage.at[cid, sid])
    npub = (n_edges + CH - 1) // CH

    def _pub(c, _):
        sl = pl.ds(c * CH, CH)
        pltpu.sync_copy(gidx_l.at[sl], stage_g.at[cid, sid, sl])
        pltpu.sync_copy(et_l.at[sl], stage_e.at[cid, sid, sl])
        pltpu.sync_copy(row_l.at[sl], stage_r.at[cid, sid, sl])
        return 0
    lax.fori_loop(0, npub, _pub, 0)

    plsc.subcore_barrier()
    pltpu.sync_copy(cstage.at[cid], cnt_all)
    cnt_diag = plsc.load_gather(cnt_all, [lane, lane])

    row_lo = sid * RPT

    def _per_src(j, _):
        nj = jnp.sum(jnp.where(lane == j, cnt_diag, 0))
        nch = (nj + CH - 1) // CH

        def _per_chunk(c, _):
            sl = pl.ds(c * CH, CH)
            pltpu.sync_copy(stage_g.at[cid, j, sl], gch)
            pltpu.sync_copy(stage_e.at[cid, j, sl], ech)
            pltpu.sync_copy(stage_r.at[cid, j, sl], rch)

            def _compact(g2, off2):
                sl2 = pl.ds(g2 * 16, 16)
                rv = rch[sl2]
                gv = gch[sl2]
                ev = ech[sl2]
                gidx_global = c * CH + g2 * 16 + lane
                m = jnp.logical_and((rv >> 5) == sid, gidx_global < nj)
                cs = plsc.cumsum(jnp.where(m, 1, 0))
                pos = off2 + cs - 1
                plsc.store_scatter(loc_g, [pos], gv, mask=m)
                plsc.store_scatter(loc_e, [pos], ev, mask=m)
                plsc.store_scatter(loc_r, [pos], rv - row_lo, mask=m)
                return off2 + jnp.sum(jnp.where(m, 1, 0))
            n_loc = lax.fori_loop(0, CH // 16, _compact, jnp.int32(0))

            nbat = (n_loc + 15) // 16

            def _batch(b, _):
                slb = pl.ds(b * 16, 16)
                gbuf[...] = loc_g[slb]
                ebuf[...] = loc_e[slb]
                pltpu.async_copy(xcat.at[gbuf], xrows, sem1).wait()
                pltpu.async_copy(rtab.at[ebuf], rrows, sem2).wait()
                nhere = n_loc - b * 16
                tvec = loc_e[slb]
                rlvec = loc_r[slb]
                cm0v = plsc.load_gather(attmu_v, [2 * tvec])
                cm1v = plsc.load_gather(attmu_v, [2 * tvec + 1])
                cs0v = plsc.load_gather(attsg_v, [2 * tvec])
                cs1v = plsc.load_gather(attsg_v, [2 * tvec + 1])

                def _edge(i):
                    rl = rlvec[i]
                    cm0 = cm0v[i]
                    cm1 = cm1v[i]
                    cs0 = cs0v[i]
                    cs1 = cs1v[i]
                    for k in range(D // 16):
                        slk = pl.ds(k * 16, 16)
                        xs = xrows[i, slk]
                        acc[rl, slk] = acc[rl, slk] + cm0 * xs
                        sk1 = pl.ds(D + k * 16, 16)
                        acc[rl, sk1] = acc[rl, sk1] + cm1 * xs
                        sk2 = pl.ds(2 * D + k * 16, 16)
                        acc[rl, sk2] = acc[rl, sk2] + cs0 * xs
                        sk3 = pl.ds(3 * D + k * 16, 16)
                        acc[rl, sk3] = acc[rl, sk3] + cs1 * xs
                    for k in range(2 * D // 16):
                        slk = pl.ds(k * 16, 16)
                        skr = pl.ds(4 * D + k * 16, 16)
                        acc[rl, skr] = acc[rl, skr] + rrows[i, slk]
                    slc = pl.ds(6 * D, 16)
                    acc[rl, slc] = acc[rl, slc] + wrow

                for i in range(16):
                    pl.when(i < nhere)(functools.partial(_edge, i))
                return 0
            lax.fori_loop(0, nbat, _batch, 0)
            return 0
        lax.fori_loop(0, nch, _per_chunk, 0)
        return 0
    lax.fori_loop(0, NS, _per_src, 0)

    pltpu.sync_copy(acc, out.at[cid, pl.ds(row_lo, RPT)])


@functools.cache
def _sc_aggregate():
  return functools.partial(
    pl.kernel,
    out_type=jax.ShapeDtypeStruct((NC, ROWS, W), jnp.float32),
    mesh=plsc.VectorSubcoreMesh(core_axis_name="c", subcore_axis_name="s",
                                num_cores=NC, num_subcores=NS),
    compiler_params=pltpu.CompilerParams(needs_layout_passes=False),
    scratch_types=[
        pltpu.VMEM((PM,), jnp.int32),
        pltpu.VMEM((C,), jnp.int32),
        pltpu.VMEM((C,), jnp.int32),
        pltpu.VMEM((C,), jnp.int32),
        pltpu.VMEM((UP,), jnp.int32),
        pltpu.VMEM((4 * N_REL,), jnp.float32),
        pltpu.VMEM((4 * N_REL,), jnp.float32),
        pltpu.VMEM((CAP,), jnp.int32),
        pltpu.VMEM((CAP,), jnp.int32),
        pltpu.VMEM((CAP,), jnp.int32),
        pltpu.VMEM((16,), jnp.int32),
        pltpu.VMEM((16, 16), jnp.int32),
        pltpu.VMEM((CH,), jnp.int32),
        pltpu.VMEM((CH,), jnp.int32),
        pltpu.VMEM((CH,), jnp.int32),
        pltpu.VMEM((CH,), jnp.int32),
        pltpu.VMEM((CH,), jnp.int32),
        pltpu.VMEM((CH,), jnp.int32),
        pltpu.VMEM((16,), jnp.int32),
        pltpu.VMEM((16,), jnp.int32),
        pltpu.VMEM((16, D), jnp.float32),
        pltpu.VMEM((16, 2 * D), jnp.float32),
        pltpu.VMEM((RPT, W), jnp.float32),
        pltpu.HBM((NC, NS, CAP), jnp.int32),
        pltpu.HBM((NC, NS, CAP), jnp.int32),
        pltpu.HBM((NC, NS, CAP), jnp.int32),
        pltpu.HBM((NC, NS, 16), jnp.int32),
        pltpu.SemaphoreType.DMA,
        pltpu.SemaphoreType.DMA,
    ],
  )(_sc_kernel_body)


def _rtab_body(re_ref, brm_ref, brs_ref, am_ref, as_ref, out_ref):
    re = re_ref[...]
    ym0 = jnp.dot(re, brm_ref[0], preferred_element_type=jnp.float32)
    ym1 = jnp.dot(re, brm_ref[1], preferred_element_type=jnp.float32)
    ys0 = jnp.dot(re, brs_ref[0], preferred_element_type=jnp.float32)
    ys1 = jnp.dot(re, brs_ref[1], preferred_element_type=jnp.float32)
    am = am_ref[...]
    asg = as_ref[...]
    ym0t = jnp.concatenate([ym0, ym0], axis=0)
    ym1t = jnp.concatenate([ym1, ym1], axis=0)
    ys0t = jnp.concatenate([ys0, ys0], axis=0)
    ys1t = jnp.concatenate([ys1, ys1], axis=0)
    rmu = am[:, 0:1] * ym0t + am[:, 1:2] * ym1t
    rsg = asg[:, 0:1] * ys0t + asg[:, 1:2] * ys1t
    out_ref[...] = jnp.concatenate([rmu, rsg], axis=1)


def _finalize_body(o_ref, tue_ref, bxm_ref, bxs_ref,
                   rm_ref, rs_ref, bm_ref, bs_ref, mu_ref, lv_ref):
    AX = o_ref[0, :, 0:4 * D] + o_ref[1, :, 0:4 * D]
    AR = o_ref[0, :, 4 * D:6 * D] + o_ref[1, :, 4 * D:6 * D]
    cnt = o_ref[0, :, 6 * D:6 * D + 1] + o_ref[1, :, 6 * D:6 * D + 1]
    inv = 1.0 / jnp.maximum(cnt, 1.0)
    tue = tue_ref[...]
    aggm = (jnp.dot(AX[:, 0:D], bxm_ref[0], preferred_element_type=jnp.float32)
            + jnp.dot(AX[:, D:2 * D], bxm_ref[1], preferred_element_type=jnp.float32)
            + AR[:, 0:D])
    aggs = (jnp.dot(AX[:, 2 * D:3 * D], bxs_ref[0], preferred_element_type=jnp.float32)
            + jnp.dot(AX[:, 3 * D:4 * D], bxs_ref[1], preferred_element_type=jnp.float32)
            + AR[:, D:2 * D])
    mu_ref[...] = (aggm * inv
                   + jnp.dot(tue, rm_ref[...], preferred_element_type=jnp.float32)
                   + bm_ref[...])
    lv_ref[...] = (aggs * inv
                   + jnp.dot(tue, rs_ref[...], preferred_element_type=jnp.float32)
                   + bs_ref[...])


@jax.jit
def _run(unseen_entity, triplets, entity_embedding, relation_embedding,
         basis_mu, att_mu, root_mu, bias_mu,
         basis_sigma, att_sigma, root_sigma, bias_sigma,
         total_unseen_entity_embedding):
    i32 = jnp.int32
    f32 = jnp.float32
    src = triplets[:, 0].astype(i32)
    rel = triplets[:, 1].astype(i32)
    dst = triplets[:, 2].astype(i32)
    npad = TP - T
    srcp = jnp.concatenate([src, jnp.full((npad,), PAD_ENT, i32)])
    relp = jnp.concatenate([rel, jnp.zeros((npad,), i32)])
    dstp = jnp.concatenate([dst, jnp.full((npad,), PAD_ENT, i32)])
    unseen = jnp.concatenate([unseen_entity.astype(i32),
                              jnp.zeros((UP - U,), i32)])
    xcat = jnp.concatenate(
        [entity_embedding, total_unseen_entity_embedding], axis=0)

    rtab = pl.pallas_call(
        _rtab_body,
        out_shape=jax.ShapeDtypeStruct((2 * N_REL, 2 * D), f32),
    )(relation_embedding, basis_mu[:, D:, :], basis_sigma[:, D:, :],
      att_mu, att_sigma)

    o = _sc_aggregate()(
        srcp, relp, dstp, unseen,
        att_mu.reshape(-1), att_sigma.reshape(-1),
        xcat, rtab,
    )

    tue_p = jnp.concatenate(
        [total_unseen_entity_embedding, jnp.zeros((ROWS - U, D), f32)], axis=0)
    mu, lv = pl.pallas_call(
        _finalize_body,
        out_shape=[jax.ShapeDtypeStruct((ROWS, D), f32),
                   jax.ShapeDtypeStruct((ROWS, D), f32)],
    )(o, tue_p, basis_mu[:, :D, :], basis_sigma[:, :D, :],
      root_mu, root_sigma, bias_mu.reshape(1, D), bias_sigma.reshape(1, D))

    mu_u = mu[:U]
    lv_u = lv[:U]
    return mu_u, mu_u, lv_u


def kernel(unseen_entity, triplets, entity_embedding, relation_embedding,
           basis_mu, att_mu, root_mu, bias_mu,
           basis_sigma, att_sigma, root_sigma, bias_sigma,
           total_unseen_entity_embedding):
    return _run(unseen_entity, triplets, entity_embedding, relation_embedding,
                basis_mu, att_mu, root_mu, bias_mu,
                basis_sigma, att_sigma, root_sigma, bias_sigma,
                total_unseen_entity_embedding)


# speedup vs baseline: 11.6310x; 1.0249x over previous
import functools

import jax
import jax.numpy as jnp
from jax import lax
from jax.experimental import pallas as pl
from jax.experimental.pallas import tpu as pltpu
from jax.experimental.pallas import tpu_sc as plsc

N_ENT = 50000
N_REL = 200
D = 128
T = 100000
U = 500

NC = 2
NS = 16
NW = NC * NS
C = 3136
TP = NW * C
G = C // 16
CAP = 6656
PM = 50016
UP = 512
ROWS = 512
RPT = ROWS // NS
W = 4 * D + 2 * D + 16
CH = 512
PAD_ENT = N_ENT


def _sc_kernel_body(srcp, relp, dstp, unseen, attmu, attsg, xcat, rtab, zpm,
                    out,
                    posmap, src_v, rel_v, dst_v, unseen_v, attmu_v, attsg_v,
                    gidx_l, et_l, row_l, cntbuf, cnt_all,
                    gch, ech, rch, loc_g, loc_e, loc_r,
                    gbuf, ebuf, xrows, rrows, acc,
                    stage_g, stage_e, stage_r, cstage, sem1, sem2):
    cid = lax.axis_index("c")
    sid = lax.axis_index("s")
    wid = sid * NC + cid
    lane = lax.iota(jnp.int32, 16)
    zeros16 = jnp.zeros((16,), jnp.int32)
    zerosf = jnp.zeros((16,), jnp.float32)
    wrow = jnp.where(lane == 0, 1.0, 0.0)

    pltpu.sync_copy(unseen.at[pl.ds(0, UP)], unseen_v)
    pltpu.sync_copy(attmu, attmu_v)
    pltpu.sync_copy(attsg, attsg_v)
    base_t = wid * C
    pltpu.sync_copy(srcp.at[pl.ds(base_t, C)], src_v)
    pltpu.sync_copy(relp.at[pl.ds(base_t, C)], rel_v)
    pltpu.sync_copy(dstp.at[pl.ds(base_t, C)], dst_v)

    pltpu.sync_copy(zpm, posmap)

    def _build_pm(j, _):
        u = unseen_v[pl.ds(j * 16, 16)]
        pos = j * 16 + lane
        plsc.store_scatter(posmap, [u], pos + 1, mask=pos < U)
        return 0
    lax.fori_loop(0, UP // 16, _build_pm, 0)

    def _zero_acc(r, _):
        for k in range(W // 16):
            acc[r, pl.ds(k * 16, 16)] = zerosf
        return 0
    lax.fori_loop(0, RPT, _zero_acc, 0)

    def _zero_loc(i, _):
        sl = pl.ds(i * 16, 16)
        loc_g[sl] = zeros16
        loc_e[sl] = zeros16
        loc_r[sl] = zeros16
        return 0
    lax.fori_loop(0, CH // 16, _zero_loc, 0)

    def _scan(g, off):
        sl = pl.ds(g * 16, 16)
        s = src_v[sl]
        r = rel_v[sl]
        d = dst_v[sl]
        pd = plsc.load_gather(posmap, [d])
        ps = plsc.load_gather(posmap, [s])
        md = pd > 0
        ms = ps > 0
        cf = plsc.cumsum(jnp.where(md, 1, 0))
        pf = off + cf - 1
        gi_f = jnp.where(ms, ps + (N_ENT - 1), s)
        plsc.store_scatter(gidx_l, [pf], gi_f, mask=md)
        plsc.store_scatter(et_l, [pf], r, mask=md)
        plsc.store_scatter(row_l, [pf], pd - 1, mask=md)
        off = off + jnp.sum(jnp.where(md, 1, 0))
        cr = plsc.cumsum(jnp.where(ms, 1, 0))
        pr = off + cr - 1
        gi_r = jnp.where(md, pd + (N_ENT - 1), d)
        plsc.store_scatter(gidx_l, [pr], gi_r, mask=ms)
        plsc.store_scatter(et_l, [pr], r + N_REL, mask=ms)
        plsc.store_scatter(row_l, [pr], ps - 1, mask=ms)
        off = off + jnp.sum(jnp.where(ms, 1, 0))
        return off
    n_edges = lax.fori_loop(0, G, _scan, jnp.int32(0))

    cntbuf[...] = zeros16 + n_edges
    pltpu.sync_copy(cntbuf, cstage.at[cid, sid])
    npub = (n_edges + CH - 1) // CH

    def _pub(c, _):
        sl = pl.ds(c * CH, CH)
        pltpu.sync_copy(gidx_l.at[sl], stage_g.at[cid, sid, sl])
        pltpu.sync_copy(et_l.at[sl], stage_e.at[cid, sid, sl])
        pltpu.sync_copy(row_l.at[sl], stage_r.at[cid, sid, sl])
        return 0
    lax.fori_loop(0, npub, _pub, 0)

    plsc.subcore_barrier()
    pltpu.sync_copy(cstage.at[cid], cnt_all)
    cnt_diag = plsc.load_gather(cnt_all, [lane, lane])

    row_lo = sid * RPT

    def _per_src(j, _):
        nj = jnp.sum(jnp.where(lane == j, cnt_diag, 0))
        nch = (nj + CH - 1) // CH

        def _per_chunk(c, _):
            sl = pl.ds(c * CH, CH)
            pltpu.sync_copy(stage_g.at[cid, j, sl], gch)
            pltpu.sync_copy(stage_e.at[cid, j, sl], ech)
            pltpu.sync_copy(stage_r.at[cid, j, sl], rch)

            def _compact(g2, off2):
                sl2 = pl.ds(g2 * 16, 16)
                rv = rch[sl2]
                gv = gch[sl2]
                ev = ech[sl2]
                gidx_global = c * CH + g2 * 16 + lane
                m = jnp.logical_and((rv >> 5) == sid, gidx_global < nj)
                cs = plsc.cumsum(jnp.where(m, 1, 0))
                pos = off2 + cs - 1
                plsc.store_scatter(loc_g, [pos], gv, mask=m)
                plsc.store_scatter(loc_e, [pos], ev, mask=m)
                plsc.store_scatter(loc_r, [pos], rv - row_lo, mask=m)
                return off2 + jnp.sum(jnp.where(m, 1, 0))
            n_loc = lax.fori_loop(0, CH // 16, _compact, jnp.int32(0))

            nbat = (n_loc + 15) // 16

            def _batch(b, _):
                slb = pl.ds(b * 16, 16)
                gbuf[...] = loc_g[slb]
                ebuf[...] = loc_e[slb]
                pltpu.async_copy(xcat.at[gbuf], xrows, sem1).wait()
                pltpu.async_copy(rtab.at[ebuf], rrows, sem2).wait()
                nhere = n_loc - b * 16
                tvec = loc_e[slb]
                rlvec = loc_r[slb]
                cm0v = plsc.load_gather(attmu_v, [2 * tvec])
                cm1v = plsc.load_gather(attmu_v, [2 * tvec + 1])
                cs0v = plsc.load_gather(attsg_v, [2 * tvec])
                cs1v = plsc.load_gather(attsg_v, [2 * tvec + 1])

                def _edge(i):
                    rl = rlvec[i]
                    cm0 = cm0v[i]
                    cm1 = cm1v[i]
                    cs0 = cs0v[i]
                    cs1 = cs1v[i]
                    for k in range(D // 16):
                        slk = pl.ds(k * 16, 16)
                        xs = xrows[i, slk]
                        acc[rl, slk] = acc[rl, slk] + cm0 * xs
                        sk1 = pl.ds(D + k * 16, 16)
                        acc[rl, sk1] = acc[rl, sk1] + cm1 * xs
                        sk2 = pl.ds(2 * D + k * 16, 16)
                        acc[rl, sk2] = acc[rl, sk2] + cs0 * xs
                        sk3 = pl.ds(3 * D + k * 16, 16)
                        acc[rl, sk3] = acc[rl, sk3] + cs1 * xs
                    for k in range(2 * D // 16):
                        slk = pl.ds(k * 16, 16)
                        skr = pl.ds(4 * D + k * 16, 16)
                        acc[rl, skr] = acc[rl, skr] + rrows[i, slk]
                    slc = pl.ds(6 * D, 16)
                    acc[rl, slc] = acc[rl, slc] + wrow

                for i in range(16):
                    pl.when(i < nhere)(functools.partial(_edge, i))
                return 0
            lax.fori_loop(0, nbat, _batch, 0)
            return 0
        lax.fori_loop(0, nch, _per_chunk, 0)
        return 0
    lax.fori_loop(0, NS, _per_src, 0)

    pltpu.sync_copy(acc, out.at[cid, pl.ds(row_lo, RPT)])


@functools.cache
def _sc_aggregate():
  return functools.partial(
    pl.kernel,
    out_type=jax.ShapeDtypeStruct((NC, ROWS, W), jnp.float32),
    mesh=plsc.VectorSubcoreMesh(core_axis_name="c", subcore_axis_name="s",
                                num_cores=NC, num_subcores=NS),
    compiler_params=pltpu.CompilerParams(needs_layout_passes=False),
    scratch_types=[
        pltpu.VMEM((PM,), jnp.int32),
        pltpu.VMEM((C,), jnp.int32),
        pltpu.VMEM((C,), jnp.int32),
        pltpu.VMEM((C,), jnp.int32),
        pltpu.VMEM((UP,), jnp.int32),
        pltpu.VMEM((4 * N_REL,), jnp.float32),
        pltpu.VMEM((4 * N_REL,), jnp.float32),
        pltpu.VMEM((CAP,), jnp.int32),
        pltpu.VMEM((CAP,), jnp.int32),
        pltpu.VMEM((CAP,), jnp.int32),
        pltpu.VMEM((16,), jnp.int32),
        pltpu.VMEM((16, 16), jnp.int32),
        pltpu.VMEM((CH,), jnp.int32),
        pltpu.VMEM((CH,), jnp.int32),
        pltpu.VMEM((CH,), jnp.int32),
        pltpu.VMEM((CH,), jnp.int32),
        pltpu.VMEM((CH,), jnp.int32),
        pltpu.VMEM((CH,), jnp.int32),
        pltpu.VMEM((16,), jnp.int32),
        pltpu.VMEM((16,), jnp.int32),
        pltpu.VMEM((16, D), jnp.float32),
        pltpu.VMEM((16, 2 * D), jnp.float32),
        pltpu.VMEM((RPT, W), jnp.float32),
        pltpu.HBM((NC, NS, CAP), jnp.int32),
        pltpu.HBM((NC, NS, CAP), jnp.int32),
        pltpu.HBM((NC, NS, CAP), jnp.int32),
        pltpu.HBM((NC, NS, 16), jnp.int32),
        pltpu.SemaphoreType.DMA,
        pltpu.SemaphoreType.DMA,
    ],
  )(_sc_kernel_body)


def _rtab_body(re_ref, brm_ref, brs_ref, am_ref, as_ref, out_ref):
    re = re_ref[...]
    ym0 = jnp.dot(re, brm_ref[0], preferred_element_type=jnp.float32)
    ym1 = jnp.dot(re, brm_ref[1], preferred_element_type=jnp.float32)
    ys0 = jnp.dot(re, brs_ref[0], preferred_element_type=jnp.float32)
    ys1 = jnp.dot(re, brs_ref[1], preferred_element_type=jnp.float32)
    am = am_ref[...]
    asg = as_ref[...]
    ym0t = jnp.concatenate([ym0, ym0], axis=0)
    ym1t = jnp.concatenate([ym1, ym1], axis=0)
    ys0t = jnp.concatenate([ys0, ys0], axis=0)
    ys1t = jnp.concatenate([ys1, ys1], axis=0)
    rmu = am[:, 0:1] * ym0t + am[:, 1:2] * ym1t
    rsg = asg[:, 0:1] * ys0t + asg[:, 1:2] * ys1t
    out_ref[...] = jnp.concatenate([rmu, rsg], axis=1)


def _finalize_body(o_ref, tue_ref, bxm_ref, bxs_ref,
                   rm_ref, rs_ref, bm_ref, bs_ref, mu_ref, lv_ref):
    AX = o_ref[0, :, 0:4 * D] + o_ref[1, :, 0:4 * D]
    AR = o_ref[0, :, 4 * D:6 * D] + o_ref[1, :, 4 * D:6 * D]
    cnt = o_ref[0, :, 6 * D:6 * D + 1] + o_ref[1, :, 6 * D:6 * D + 1]
    inv = 1.0 / jnp.maximum(cnt, 1.0)
    tue = tue_ref[...]
    aggm = (jnp.dot(AX[:, 0:D], bxm_ref[0], preferred_element_type=jnp.float32)
            + jnp.dot(AX[:, D:2 * D], bxm_ref[1], preferred_element_type=jnp.float32)
            + AR[:, 0:D])
    aggs = (jnp.dot(AX[:, 2 * D:3 * D], bxs_ref[0], preferred_element_type=jnp.float32)
            + jnp.dot(AX[:, 3 * D:4 * D], bxs_ref[1], preferred_element_type=jnp.float32)
            + AR[:, D:2 * D])
    mu_ref[...] = (aggm * inv
                   + jnp.dot(tue, rm_ref[...], preferred_element_type=jnp.float32)
                   + bm_ref[...])
    lv_ref[...] = (aggs * inv
                   + jnp.dot(tue, rs_ref[...], preferred_element_type=jnp.float32)
                   + bs_ref[...])


@jax.jit
def _run(unseen_entity, triplets, entity_embedding, relation_embedding,
         basis_mu, att_mu, root_mu, bias_mu,
         basis_sigma, att_sigma, root_sigma, bias_sigma,
         total_unseen_entity_embedding):
    i32 = jnp.int32
    f32 = jnp.float32
    src = triplets[:, 0].astype(i32)
    rel = triplets[:, 1].astype(i32)
    dst = triplets[:, 2].astype(i32)
    npad = TP - T
    srcp = jnp.concatenate([src, jnp.full((npad,), PAD_ENT, i32)])
    relp = jnp.concatenate([rel, jnp.zeros((npad,), i32)])
    dstp = jnp.concatenate([dst, jnp.full((npad,), PAD_ENT, i32)])
    unseen = jnp.concatenate([unseen_entity.astype(i32),
                              jnp.zeros((UP - U,), i32)])
    xcat = jnp.concatenate(
        [entity_embedding, total_unseen_entity_embedding], axis=0)

    rtab = pl.pallas_call(
        _rtab_body,
        out_shape=jax.ShapeDtypeStruct((2 * N_REL, 2 * D), f32),
    )(relation_embedding, basis_mu[:, D:, :], basis_sigma[:, D:, :],
      att_mu, att_sigma)

    o = _sc_aggregate()(
        srcp, relp, dstp, unseen,
        att_mu.reshape(-1), att_sigma.reshape(-1),
        xcat, rtab, jnp.zeros((PM,), i32),
    )

    tue_p = jnp.concatenate(
        [total_unseen_entity_embedding, jnp.zeros((ROWS - U, D), f32)], axis=0)
    mu, lv = pl.pallas_call(
        _finalize_body,
        out_shape=[jax.ShapeDtypeStruct((ROWS, D), f32),
                   jax.ShapeDtypeStruct((ROWS, D), f32)],
    )(o, tue_p, basis_mu[:, :D, :], basis_sigma[:, :D, :],
      root_mu, root_sigma, bias_mu.reshape(1, D), bias_sigma.reshape(1, D))

    mu_u = mu[:U]
    lv_u = lv[:U]
    return mu_u, mu_u, lv_u


def kernel(unseen_entity, triplets, entity_embedding, relation_embedding,
           basis_mu, att_mu, root_mu, bias_mu,
           basis_sigma, att_sigma, root_sigma, bias_sigma,
           total_unseen_entity_embedding):
    return _run(unseen_entity, triplets, entity_embedding, relation_embedding,
                basis_mu, att_mu, root_mu, bias_mu,
                basis_sigma, att_sigma, root_sigma, bias_sigma,
                total_unseen_entity_embedding)


# speedup vs baseline: 11.8327x; 1.0173x over previous
import functools

import jax
import jax.numpy as jnp
from jax import lax
from jax.experimental import pallas as pl
from jax.experimental.pallas import tpu as pltpu
from jax.experimental.pallas import tpu_sc as plsc

N_ENT = 50000
N_REL = 200
D = 128
T = 100000
U = 500

NC = 2
NS = 16
NW = NC * NS
C = 3136
TP = NW * C
G = C // 16
CAP = 6656
PM = 50016
UP = 512
ROWS = 512
RPT = ROWS // NS
W = 4 * D + 2 * D + 16
CH = 512
PAD_ENT = N_ENT


def _sc_kernel_body(srcp, relp, dstp, unseen, attmu, attsg, xcat, rtab, zpm,
                    out,
                    posmap, src_v, rel_v, dst_v, unseen_v, attmu_v, attsg_v,
                    gidx_l, et_l, row_l, cntbuf, cnt_all,
                    gch, ech, rch, loc_g, loc_e, loc_r,
                    gbuf, ebuf, xrows, rrows, acc,
                    stage_g, stage_e, stage_r, cstage, sem1, sem2):
    cid = lax.axis_index("c")
    sid = lax.axis_index("s")
    wid = sid * NC + cid
    lane = lax.iota(jnp.int32, 16)
    zeros16 = jnp.zeros((16,), jnp.int32)
    zerosf = jnp.zeros((16,), jnp.float32)
    wrow = jnp.where(lane == 0, 1.0, 0.0)

    pltpu.sync_copy(unseen.at[pl.ds(0, UP)], unseen_v)
    pltpu.sync_copy(attmu, attmu_v)
    pltpu.sync_copy(attsg, attsg_v)
    base_t = wid * C
    pltpu.sync_copy(srcp.at[pl.ds(base_t, C)], src_v)
    pltpu.sync_copy(relp.at[pl.ds(base_t, C)], rel_v)
    pltpu.sync_copy(dstp.at[pl.ds(base_t, C)], dst_v)

    pltpu.sync_copy(zpm, posmap)

    def _build_pm(j, _):
        u = unseen_v[pl.ds(j * 16, 16)]
        pos = j * 16 + lane
        plsc.store_scatter(posmap, [u], pos + 1, mask=pos < U)
        return 0
    lax.fori_loop(0, UP // 16, _build_pm, 0)

    def _zero_acc(r, _):
        for k in range(W // 16):
            acc[r, pl.ds(k * 16, 16)] = zerosf
        return 0
    lax.fori_loop(0, RPT, _zero_acc, 0)

    def _zero_loc(i, _):
        sl = pl.ds(i * 16, 16)
        loc_g[sl] = zeros16
        loc_e[sl] = zeros16
        loc_r[sl] = zeros16
        return 0
    lax.fori_loop(0, CH // 16, _zero_loc, 0)

    def _scan(g, off):
        sl = pl.ds(g * 16, 16)
        s = src_v[sl]
        r = rel_v[sl]
        d = dst_v[sl]
        pd = plsc.load_gather(posmap, [d])
        ps = plsc.load_gather(posmap, [s])
        md = pd > 0
        ms = ps > 0
        cf = plsc.cumsum(jnp.where(md, 1, 0))
        pf = off + cf - 1
        gi_f = jnp.where(ms, ps + (N_ENT - 1), s)
        plsc.store_scatter(gidx_l, [pf], gi_f, mask=md)
        plsc.store_scatter(et_l, [pf], r, mask=md)
        plsc.store_scatter(row_l, [pf], pd - 1, mask=md)
        off = off + jnp.sum(jnp.where(md, 1, 0))
        cr = plsc.cumsum(jnp.where(ms, 1, 0))
        pr = off + cr - 1
        gi_r = jnp.where(md, pd + (N_ENT - 1), d)
        plsc.store_scatter(gidx_l, [pr], gi_r, mask=ms)
        plsc.store_scatter(et_l, [pr], r + N_REL, mask=ms)
        plsc.store_scatter(row_l, [pr], ps - 1, mask=ms)
        off = off + jnp.sum(jnp.where(ms, 1, 0))
        return off
    n_edges = lax.fori_loop(0, G, _scan, jnp.int32(0))

    cntbuf[...] = zeros16 + n_edges
    pltpu.sync_copy(cntbuf, cstage.at[cid, sid])
    npub = (n_edges + CH - 1) // CH

    def _pub(c, _):
        sl = pl.ds(c * CH, CH)
        pltpu.sync_copy(gidx_l.at[sl], stage_g.at[cid, sid, sl])
        pltpu.sync_copy(et_l.at[sl], stage_e.at[cid, sid, sl])
        pltpu.sync_copy(row_l.at[sl], stage_r.at[cid, sid, sl])
        return 0
    lax.fori_loop(0, npub, _pub, 0)

    plsc.subcore_barrier()
    pltpu.sync_copy(cstage.at[cid], cnt_all)
    cnt_diag = plsc.load_gather(cnt_all, [lane, lane])

    row_lo = sid * RPT

    def _per_src(j, _):
        nj = jnp.sum(jnp.where(lane == j, cnt_diag, 0))
        nch = (nj + CH - 1) // CH

        def _per_chunk(c, _):
            sl = pl.ds(c * CH, CH)
            pltpu.sync_copy(stage_g.at[cid, j, sl], gch)
            pltpu.sync_copy(stage_e.at[cid, j, sl], ech)
            pltpu.sync_copy(stage_r.at[cid, j, sl], rch)

            def _compact(g2, off2):
                sl2 = pl.ds(g2 * 16, 16)
                rv = rch[sl2]
                gv = gch[sl2]
                ev = ech[sl2]
                gidx_global = c * CH + g2 * 16 + lane
                m = jnp.logical_and((rv >> 5) == sid, gidx_global < nj)
                cs = plsc.cumsum(jnp.where(m, 1, 0))
                pos = off2 + cs - 1
                plsc.store_scatter(loc_g, [pos], gv, mask=m)
                plsc.store_scatter(loc_e, [pos], ev, mask=m)
                plsc.store_scatter(loc_r, [pos], rv - row_lo, mask=m)
                return off2 + jnp.sum(jnp.where(m, 1, 0))
            n_loc = lax.fori_loop(0, CH // 16, _compact, jnp.int32(0))

            nbat = (n_loc + 15) // 16

            def _batch(b, _):
                slb = pl.ds(b * 16, 16)
                gbuf[...] = loc_g[slb]
                ebuf[...] = loc_e[slb]
                dx = pltpu.async_copy(xcat.at[gbuf], xrows, sem1)
                dr = pltpu.async_copy(rtab.at[ebuf], rrows, sem2)
                dx.wait()
                dr.wait()
                nhere = n_loc - b * 16
                tvec = loc_e[slb]
                rlvec = loc_r[slb]
                cm0v = plsc.load_gather(attmu_v, [2 * tvec])
                cm1v = plsc.load_gather(attmu_v, [2 * tvec + 1])
                cs0v = plsc.load_gather(attsg_v, [2 * tvec])
                cs1v = plsc.load_gather(attsg_v, [2 * tvec + 1])

                def _edge(i):
                    rl = rlvec[i]
                    cm0 = cm0v[i]
                    cm1 = cm1v[i]
                    cs0 = cs0v[i]
                    cs1 = cs1v[i]
                    for k in range(D // 16):
                        slk = pl.ds(k * 16, 16)
                        xs = xrows[i, slk]
                        acc[rl, slk] = acc[rl, slk] + cm0 * xs
                        sk1 = pl.ds(D + k * 16, 16)
                        acc[rl, sk1] = acc[rl, sk1] + cm1 * xs
                        sk2 = pl.ds(2 * D + k * 16, 16)
                        acc[rl, sk2] = acc[rl, sk2] + cs0 * xs
                        sk3 = pl.ds(3 * D + k * 16, 16)
                        acc[rl, sk3] = acc[rl, sk3] + cs1 * xs
                    for k in range(2 * D // 16):
                        slk = pl.ds(k * 16, 16)
                        skr = pl.ds(4 * D + k * 16, 16)
                        acc[rl, skr] = acc[rl, skr] + rrows[i, slk]
                    slc = pl.ds(6 * D, 16)
                    acc[rl, slc] = acc[rl, slc] + wrow

                for i in range(16):
                    pl.when(i < nhere)(functools.partial(_edge, i))
                return 0
            lax.fori_loop(0, nbat, _batch, 0)
            return 0
        lax.fori_loop(0, nch, _per_chunk, 0)
        return 0
    lax.fori_loop(0, NS, _per_src, 0)

    pltpu.sync_copy(acc, out.at[cid, pl.ds(row_lo, RPT)])


@functools.cache
def _sc_aggregate():
  return functools.partial(
    pl.kernel,
    out_type=jax.ShapeDtypeStruct((NC, ROWS, W), jnp.float32),
    mesh=plsc.VectorSubcoreMesh(core_axis_name="c", subcore_axis_name="s",
                                num_cores=NC, num_subcores=NS),
    compiler_params=pltpu.CompilerParams(needs_layout_passes=False),
    scratch_types=[
        pltpu.VMEM((PM,), jnp.int32),
        pltpu.VMEM((C,), jnp.int32),
        pltpu.VMEM((C,), jnp.int32),
        pltpu.VMEM((C,), jnp.int32),
        pltpu.VMEM((UP,), jnp.int32),
        pltpu.VMEM((4 * N_REL,), jnp.float32),
        pltpu.VMEM((4 * N_REL,), jnp.float32),
        pltpu.VMEM((CAP,), jnp.int32),
        pltpu.VMEM((CAP,), jnp.int32),
        pltpu.VMEM((CAP,), jnp.int32),
        pltpu.VMEM((16,), jnp.int32),
        pltpu.VMEM((16, 16), jnp.int32),
        pltpu.VMEM((CH,), jnp.int32),
        pltpu.VMEM((CH,), jnp.int32),
        pltpu.VMEM((CH,), jnp.int32),
        pltpu.VMEM((CH,), jnp.int32),
        pltpu.VMEM((CH,), jnp.int32),
        pltpu.VMEM((CH,), jnp.int32),
        pltpu.VMEM((16,), jnp.int32),
        pltpu.VMEM((16,), jnp.int32),
        pltpu.VMEM((16, D), jnp.float32),
        pltpu.VMEM((16, 2 * D), jnp.float32),
        pltpu.VMEM((RPT, W), jnp.float32),
        pltpu.HBM((NC, NS, CAP), jnp.int32),
        pltpu.HBM((NC, NS, CAP), jnp.int32),
        pltpu.HBM((NC, NS, CAP), jnp.int32),
        pltpu.HBM((NC, NS, 16), jnp.int32),
        pltpu.SemaphoreType.DMA,
        pltpu.SemaphoreType.DMA,
    ],
  )(_sc_kernel_body)


def _rtab_body(re_ref, brm_ref, brs_ref, am_ref, as_ref, out_ref):
    re = re_ref[...]
    ym0 = jnp.dot(re, brm_ref[0], preferred_element_type=jnp.float32)
    ym1 = jnp.dot(re, brm_ref[1], preferred_element_type=jnp.float32)
    ys0 = jnp.dot(re, brs_ref[0], preferred_element_type=jnp.float32)
    ys1 = jnp.dot(re, brs_ref[1], preferred_element_type=jnp.float32)
    am = am_ref[...]
    asg = as_ref[...]
    ym0t = jnp.concatenate([ym0, ym0], axis=0)
    ym1t = jnp.concatenate([ym1, ym1], axis=0)
    ys0t = jnp.concatenate([ys0, ys0], axis=0)
    ys1t = jnp.concatenate([ys1, ys1], axis=0)
    rmu = am[:, 0:1] * ym0t + am[:, 1:2] * ym1t
    rsg = asg[:, 0:1] * ys0t + asg[:, 1:2] * ys1t
    out_ref[...] = jnp.concatenate([rmu, rsg], axis=1)


def _finalize_body(o_ref, tue_ref, bxm_ref, bxs_ref,
                   rm_ref, rs_ref, bm_ref, bs_ref, mu_ref, lv_ref):
    AX = o_ref[0, :, 0:4 * D] + o_ref[1, :, 0:4 * D]
    AR = o_ref[0, :, 4 * D:6 * D] + o_ref[1, :, 4 * D:6 * D]
    cnt = o_ref[0, :, 6 * D:6 * D + 1] + o_ref[1, :, 6 * D:6 * D + 1]
    inv = 1.0 / jnp.maximum(cnt, 1.0)
    tue = tue_ref[...]
    aggm = (jnp.dot(AX[:, 0:D], bxm_ref[0], preferred_element_type=jnp.float32)
            + jnp.dot(AX[:, D:2 * D], bxm_ref[1], preferred_element_type=jnp.float32)
            + AR[:, 0:D])
    aggs = (jnp.dot(AX[:, 2 * D:3 * D], bxs_ref[0], preferred_element_type=jnp.float32)
            + jnp.dot(AX[:, 3 * D:4 * D], bxs_ref[1], preferred_element_type=jnp.float32)
            + AR[:, D:2 * D])
    mu_ref[...] = (aggm * inv
                   + jnp.dot(tue, rm_ref[...], preferred_element_type=jnp.float32)
                   + bm_ref[...])
    lv_ref[...] = (aggs * inv
                   + jnp.dot(tue, rs_ref[...], preferred_element_type=jnp.float32)
                   + bs_ref[...])


@jax.jit
def _run(unseen_entity, triplets, entity_embedding, relation_embedding,
         basis_mu, att_mu, root_mu, bias_mu,
         basis_sigma, att_sigma, root_sigma, bias_sigma,
         total_unseen_entity_embedding):
    i32 = jnp.int32
    f32 = jnp.float32
    src = triplets[:, 0].astype(i32)
    rel = triplets[:, 1].astype(i32)
    dst = triplets[:, 2].astype(i32)
    npad = TP - T
    srcp = jnp.concatenate([src, jnp.full((npad,), PAD_ENT, i32)])
    relp = jnp.concatenate([rel, jnp.zeros((npad,), i32)])
    dstp = jnp.concatenate([dst, jnp.full((npad,), PAD_ENT, i32)])
    unseen = jnp.concatenate([unseen_entity.astype(i32),
                              jnp.zeros((UP - U,), i32)])
    xcat = jnp.concatenate(
        [entity_embedding, total_unseen_entity_embedding], axis=0)

    rtab = pl.pallas_call(
        _rtab_body,
        out_shape=jax.ShapeDtypeStruct((2 * N_REL, 2 * D), f32),
    )(relation_embedding, basis_mu[:, D:, :], basis_sigma[:, D:, :],
      att_mu, att_sigma)

    o = _sc_aggregate()(
        srcp, relp, dstp, unseen,
        att_mu.reshape(-1), att_sigma.reshape(-1),
        xcat, rtab, jnp.zeros((PM,), i32),
    )

    tue_p = jnp.concatenate(
        [total_unseen_entity_embedding, jnp.zeros((ROWS - U, D), f32)], axis=0)
    mu, lv = pl.pallas_call(
        _finalize_body,
        out_shape=[jax.ShapeDtypeStruct((ROWS, D), f32),
                   jax.ShapeDtypeStruct((ROWS, D), f32)],
    )(o, tue_p, basis_mu[:, :D, :], basis_sigma[:, :D, :],
      root_mu, root_sigma, bias_mu.reshape(1, D), bias_sigma.reshape(1, D))

    mu_u = mu[:U]
    lv_u = lv[:U]
    return mu_u, mu_u, lv_u


def kernel(unseen_entity, triplets, entity_embedding, relation_embedding,
           basis_mu, att_mu, root_mu, bias_mu,
           basis_sigma, att_sigma, root_sigma, bias_sigma,
           total_unseen_entity_embedding):
    return _run(unseen_entity, triplets, entity_embedding, relation_embedding,
                basis_mu, att_mu, root_mu, bias_mu,
                basis_sigma, att_sigma, root_sigma, bias_sigma,
                total_unseen_entity_embedding)


# speedup vs baseline: 11.8483x; 1.0013x over previous
import functools

import jax
import jax.numpy as jnp
from jax import lax
from jax.experimental import pallas as pl
from jax.experimental.pallas import tpu as pltpu
from jax.experimental.pallas import tpu_sc as plsc

N_ENT = 50000
N_REL = 200
D = 128
T = 100000
U = 500

NC = 2
NS = 16
NW = NC * NS
C = 3136
TP = NW * C
G = C // 16
CAP = 6656
PM = 50016
UP = 512
ROWS = 512
RPT = ROWS // NS
W = 4 * D + 2 * D + 16
CH = 512
PAD_ENT = N_ENT


def _sc_kernel_body(srcp, relp, dstp, unseen, attmu, attsg, xcat, rtab, zpm,
                    out,
                    posmap, src_v, rel_v, dst_v, unseen_v, attmu_v, attsg_v,
                    gidx_l, et_l, row_l, cntbuf, cnt_all,
                    gch, ech, rch, loc_g, loc_e, loc_r,
                    gbuf, ebuf, xrows, rrows, acc,
                    stage_g, stage_e, stage_r, cstage, sem1, sem2):
    cid = lax.axis_index("c")
    sid = lax.axis_index("s")
    wid = sid * NC + cid
    lane = lax.iota(jnp.int32, 16)
    zeros16 = jnp.zeros((16,), jnp.int32)
    zerosf = jnp.zeros((16,), jnp.float32)
    wrow = jnp.where(lane == 0, 1.0, 0.0)

    pltpu.sync_copy(unseen.at[pl.ds(0, UP)], unseen_v)
    pltpu.sync_copy(attmu, attmu_v)
    pltpu.sync_copy(attsg, attsg_v)
    base_t = wid * C
    pltpu.sync_copy(srcp.at[pl.ds(base_t, C)], src_v)
    pltpu.sync_copy(relp.at[pl.ds(base_t, C)], rel_v)
    pltpu.sync_copy(dstp.at[pl.ds(base_t, C)], dst_v)

    pltpu.sync_copy(zpm, posmap)

    def _build_pm(j, _):
        u = unseen_v[pl.ds(j * 16, 16)]
        pos = j * 16 + lane
        plsc.store_scatter(posmap, [u], pos + 1, mask=pos < U)
        return 0
    lax.fori_loop(0, UP // 16, _build_pm, 0)

    def _zero_acc(r, _):
        for k in range(W // 16):
            acc[r, pl.ds(k * 16, 16)] = zerosf
        return 0
    lax.fori_loop(0, RPT, _zero_acc, 0)

    def _zero_loc(i, _):
        sl = pl.ds(i * 16, 16)
        loc_g[sl] = zeros16
        loc_e[sl] = zeros16
        loc_r[sl] = zeros16
        return 0
    lax.fori_loop(0, CH // 16, _zero_loc, 0)

    def _scan(g, off):
        sl = pl.ds(g * 16, 16)
        s = src_v[sl]
        r = rel_v[sl]
        d = dst_v[sl]
        pd = plsc.load_gather(posmap, [d])
        ps = plsc.load_gather(posmap, [s])
        md = pd > 0
        ms = ps > 0
        cf = plsc.cumsum(jnp.where(md, 1, 0))
        pf = off + cf - 1
        gi_f = jnp.where(ms, ps + (N_ENT - 1), s)
        plsc.store_scatter(gidx_l, [pf], gi_f, mask=md)
        plsc.store_scatter(et_l, [pf], r, mask=md)
        plsc.store_scatter(row_l, [pf], pd - 1, mask=md)
        off = off + cf[15]
        cr = plsc.cumsum(jnp.where(ms, 1, 0))
        pr = off + cr - 1
        gi_r = jnp.where(md, pd + (N_ENT - 1), d)
        plsc.store_scatter(gidx_l, [pr], gi_r, mask=ms)
        plsc.store_scatter(et_l, [pr], r + N_REL, mask=ms)
        plsc.store_scatter(row_l, [pr], ps - 1, mask=ms)
        off = off + cr[15]
        return off
    n_edges = lax.fori_loop(0, G, _scan, jnp.int32(0))

    cntbuf[...] = zeros16 + n_edges
    pltpu.sync_copy(cntbuf, cstage.at[cid, sid])
    npub = (n_edges + CH - 1) // CH

    def _pub(c, _):
        sl = pl.ds(c * CH, CH)
        pltpu.sync_copy(gidx_l.at[sl], stage_g.at[cid, sid, sl])
        pltpu.sync_copy(et_l.at[sl], stage_e.at[cid, sid, sl])
        pltpu.sync_copy(row_l.at[sl], stage_r.at[cid, sid, sl])
        return 0
    lax.fori_loop(0, npub, _pub, 0)

    plsc.subcore_barrier()
    pltpu.sync_copy(cstage.at[cid], cnt_all)
    cnt_diag = plsc.load_gather(cnt_all, [lane, lane])

    row_lo = sid * RPT

    def _per_src(j, _):
        nj = jnp.sum(jnp.where(lane == j, cnt_diag, 0))
        nch = (nj + CH - 1) // CH

        def _per_chunk(c, _):
            sl = pl.ds(c * CH, CH)
            pltpu.sync_copy(stage_g.at[cid, j, sl], gch)
            pltpu.sync_copy(stage_e.at[cid, j, sl], ech)
            pltpu.sync_copy(stage_r.at[cid, j, sl], rch)

            def _compact(g2, off2):
                sl2 = pl.ds(g2 * 16, 16)
                rv = rch[sl2]
                gv = gch[sl2]
                ev = ech[sl2]
                gidx_global = c * CH + g2 * 16 + lane
                m = jnp.logical_and((rv >> 5) == sid, gidx_global < nj)
                cs = plsc.cumsum(jnp.where(m, 1, 0))
                pos = off2 + cs - 1
                plsc.store_scatter(loc_g, [pos], gv, mask=m)
                plsc.store_scatter(loc_e, [pos], ev, mask=m)
                plsc.store_scatter(loc_r, [pos], rv - row_lo, mask=m)
                return off2 + cs[15]
            n_loc = lax.fori_loop(0, CH // 16, _compact, jnp.int32(0))

            nbat = (n_loc + 15) // 16

            def _batch(b, _):
                slb = pl.ds(b * 16, 16)
                gbuf[...] = loc_g[slb]
                ebuf[...] = loc_e[slb]
                dx = pltpu.async_copy(xcat.at[gbuf], xrows, sem1)
                dr = pltpu.async_copy(rtab.at[ebuf], rrows, sem2)
                dx.wait()
                dr.wait()
                nhere = n_loc - b * 16
                tvec = loc_e[slb]
                rlvec = loc_r[slb]
                cm0v = plsc.load_gather(attmu_v, [2 * tvec])
                cm1v = plsc.load_gather(attmu_v, [2 * tvec + 1])
                cs0v = plsc.load_gather(attsg_v, [2 * tvec])
                cs1v = plsc.load_gather(attsg_v, [2 * tvec + 1])

                def _edge(i):
                    rl = rlvec[i]
                    cm0 = cm0v[i]
                    cm1 = cm1v[i]
                    cs0 = cs0v[i]
                    cs1 = cs1v[i]
                    for k in range(D // 16):
                        slk = pl.ds(k * 16, 16)
                        xs = xrows[i, slk]
                        acc[rl, slk] = acc[rl, slk] + cm0 * xs
                        sk1 = pl.ds(D + k * 16, 16)
                        acc[rl, sk1] = acc[rl, sk1] + cm1 * xs
                        sk2 = pl.ds(2 * D + k * 16, 16)
                        acc[rl, sk2] = acc[rl, sk2] + cs0 * xs
                        sk3 = pl.ds(3 * D + k * 16, 16)
                        acc[rl, sk3] = acc[rl, sk3] + cs1 * xs
                    for k in range(2 * D // 16):
                        slk = pl.ds(k * 16, 16)
                        skr = pl.ds(4 * D + k * 16, 16)
                        acc[rl, skr] = acc[rl, skr] + rrows[i, slk]
                    slc = pl.ds(6 * D, 16)
                    acc[rl, slc] = acc[rl, slc] + wrow

                for i in range(16):
                    pl.when(i < nhere)(functools.partial(_edge, i))
                return 0
            lax.fori_loop(0, nbat, _batch, 0)
            return 0
        lax.fori_loop(0, nch, _per_chunk, 0)
        return 0
    lax.fori_loop(0, NS, _per_src, 0)

    pltpu.sync_copy(acc, out.at[cid, pl.ds(row_lo, RPT)])


@functools.cache
def _sc_aggregate():
  return functools.partial(
    pl.kernel,
    out_type=jax.ShapeDtypeStruct((NC, ROWS, W), jnp.float32),
    mesh=plsc.VectorSubcoreMesh(core_axis_name="c", subcore_axis_name="s",
                                num_cores=NC, num_subcores=NS),
    compiler_params=pltpu.CompilerParams(needs_layout_passes=False),
    scratch_types=[
        pltpu.VMEM((PM,), jnp.int32),
        pltpu.VMEM((C,), jnp.int32),
        pltpu.VMEM((C,), jnp.int32),
        pltpu.VMEM((C,), jnp.int32),
        pltpu.VMEM((UP,), jnp.int32),
        pltpu.VMEM((4 * N_REL,), jnp.float32),
        pltpu.VMEM((4 * N_REL,), jnp.float32),
        pltpu.VMEM((CAP,), jnp.int32),
        pltpu.VMEM((CAP,), jnp.int32),
        pltpu.VMEM((CAP,), jnp.int32),
        pltpu.VMEM((16,), jnp.int32),
        pltpu.VMEM((16, 16), jnp.int32),
        pltpu.VMEM((CH,), jnp.int32),
        pltpu.VMEM((CH,), jnp.int32),
        pltpu.VMEM((CH,), jnp.int32),
        pltpu.VMEM((CH,), jnp.int32),
        pltpu.VMEM((CH,), jnp.int32),
        pltpu.VMEM((CH,), jnp.int32),
        pltpu.VMEM((16,), jnp.int32),
        pltpu.VMEM((16,), jnp.int32),
        pltpu.VMEM((16, D), jnp.float32),
        pltpu.VMEM((16, 2 * D), jnp.float32),
        pltpu.VMEM((RPT, W), jnp.float32),
        pltpu.HBM((NC, NS, CAP), jnp.int32),
        pltpu.HBM((NC, NS, CAP), jnp.int32),
        pltpu.HBM((NC, NS, CAP), jnp.int32),
        pltpu.HBM((NC, NS, 16), jnp.int32),
        pltpu.SemaphoreType.DMA,
        pltpu.SemaphoreType.DMA,
    ],
  )(_sc_kernel_body)


def _rtab_body(re_ref, brm_ref, brs_ref, am_ref, as_ref, out_ref):
    re = re_ref[...]
    ym0 = jnp.dot(re, brm_ref[0], preferred_element_type=jnp.float32)
    ym1 = jnp.dot(re, brm_ref[1], preferred_element_type=jnp.float32)
    ys0 = jnp.dot(re, brs_ref[0], preferred_element_type=jnp.float32)
    ys1 = jnp.dot(re, brs_ref[1], preferred_element_type=jnp.float32)
    am = am_ref[...]
    asg = as_ref[...]
    ym0t = jnp.concatenate([ym0, ym0], axis=0)
    ym1t = jnp.concatenate([ym1, ym1], axis=0)
    ys0t = jnp.concatenate([ys0, ys0], axis=0)
    ys1t = jnp.concatenate([ys1, ys1], axis=0)
    rmu = am[:, 0:1] * ym0t + am[:, 1:2] * ym1t
    rsg = asg[:, 0:1] * ys0t + asg[:, 1:2] * ys1t
    out_ref[...] = jnp.concatenate([rmu, rsg], axis=1)


def _finalize_body(o_ref, tue_ref, bxm_ref, bxs_ref,
                   rm_ref, rs_ref, bm_ref, bs_ref, mu_ref, lv_ref):
    AX = o_ref[0, :, 0:4 * D] + o_ref[1, :, 0:4 * D]
    AR = o_ref[0, :, 4 * D:6 * D] + o_ref[1, :, 4 * D:6 * D]
    cnt = o_ref[0, :, 6 * D:6 * D + 1] + o_ref[1, :, 6 * D:6 * D + 1]
    inv = 1.0 / jnp.maximum(cnt, 1.0)
    tue = tue_ref[...]
    aggm = (jnp.dot(AX[:, 0:D], bxm_ref[0], preferred_element_type=jnp.float32)
            + jnp.dot(AX[:, D:2 * D], bxm_ref[1], preferred_element_type=jnp.float32)
            + AR[:, 0:D])
    aggs = (jnp.dot(AX[:, 2 * D:3 * D], bxs_ref[0], preferred_element_type=jnp.float32)
            + jnp.dot(AX[:, 3 * D:4 * D], bxs_ref[1], preferred_element_type=jnp.float32)
            + AR[:, D:2 * D])
    mu_ref[...] = (aggm * inv
                   + jnp.dot(tue, rm_ref[...], preferred_element_type=jnp.float32)
                   + bm_ref[...])
    lv_ref[...] = (aggs * inv
                   + jnp.dot(tue, rs_ref[...], preferred_element_type=jnp.float32)
                   + bs_ref[...])


@jax.jit
def _run(unseen_entity, triplets, entity_embedding, relation_embedding,
         basis_mu, att_mu, root_mu, bias_mu,
         basis_sigma, att_sigma, root_sigma, bias_sigma,
         total_unseen_entity_embedding):
    i32 = jnp.int32
    f32 = jnp.float32
    src = triplets[:, 0].astype(i32)
    rel = triplets[:, 1].astype(i32)
    dst = triplets[:, 2].astype(i32)
    npad = TP - T
    srcp = jnp.concatenate([src, jnp.full((npad,), PAD_ENT, i32)])
    relp = jnp.concatenate([rel, jnp.zeros((npad,), i32)])
    dstp = jnp.concatenate([dst, jnp.full((npad,), PAD_ENT, i32)])
    unseen = jnp.concatenate([unseen_entity.astype(i32),
                              jnp.zeros((UP - U,), i32)])
    xcat = jnp.concatenate(
        [entity_embedding, total_unseen_entity_embedding], axis=0)

    rtab = pl.pallas_call(
        _rtab_body,
        out_shape=jax.ShapeDtypeStruct((2 * N_REL, 2 * D), f32),
    )(relation_embedding, basis_mu[:, D:, :], basis_sigma[:, D:, :],
      att_mu, att_sigma)

    o = _sc_aggregate()(
        srcp, relp, dstp, unseen,
        att_mu.reshape(-1), att_sigma.reshape(-1),
        xcat, rtab, jnp.zeros((PM,), i32),
    )

    tue_p = jnp.concatenate(
        [total_unseen_entity_embedding, jnp.zeros((ROWS - U, D), f32)], axis=0)
    mu, lv = pl.pallas_call(
        _finalize_body,
        out_shape=[jax.ShapeDtypeStruct((ROWS, D), f32),
                   jax.ShapeDtypeStruct((ROWS, D), f32)],
    )(o, tue_p, basis_mu[:, :D, :], basis_sigma[:, :D, :],
      root_mu, root_sigma, bias_mu.reshape(1, D), bias_sigma.reshape(1, D))

    mu_u = mu[:U]
    lv_u = lv[:U]
    return mu_u, mu_u, lv_u


def kernel(unseen_entity, triplets, entity_embedding, relation_embedding,
           basis_mu, att_mu, root_mu, bias_mu,
           basis_sigma, att_sigma, root_sigma, bias_sigma,
           total_unseen_entity_embedding):
    return _run(unseen_entity, triplets, entity_embedding, relation_embedding,
                basis_mu, att_mu, root_mu, bias_mu,
                basis_sigma, att_sigma, root_sigma, bias_sigma,
                total_unseen_entity_embedding)
